# Initial kernel scaffold; baseline (speedup 1.0000x reference)
import jax
import jax.numpy as jnp
from jax.experimental import pallas as pl


def kernel(x, Wd, Wg, We, be, Wu, gamma):
    raise NotImplementedError("write your pallas kernel here")


# speedup vs baseline: 1.9201x; 1.9201x over previous
import functools

import jax
import jax.numpy as jnp
from jax.experimental import pallas as pl
from jax.experimental.pallas import tpu as pltpu

B, N, IN = 4, 2048, 1024
R = 256
E = 8
K = 2
OUT = 1024
TEMP = 1.0

TBLK = 512


def _fused_kernel(x_ref, wd_ref, wg_ref, we_ref, be_ref, wu_ref, gamma_ref,
                  out_ref):
    i = pl.program_id(0)
    xb = x_ref[...]
    hp = jax.lax.dot_general(xb, wd_ref[...], (((1,), (1,)), ((), ())),
                             preferred_element_type=jnp.float32)
    h = 0.5 * hp * (1.0 + jax.lax.erf(hp * 0.7071067811865476))

    logits = jax.lax.dot_general(h, wg_ref[...], (((1,), (1,)), ((), ())),
                                 preferred_element_type=jnp.float32)
    lmax = jnp.max(logits, axis=-1, keepdims=True)
    u = jnp.exp((logits - lmax) / TEMP)
    eidx = jax.lax.broadcasted_iota(jnp.int32, u.shape, 1)
    m1 = jnp.max(u, axis=-1, keepdims=True)
    idx1 = jnp.min(jnp.where(u == m1, eidx, E), axis=-1, keepdims=True)
    sel1 = eidx == idx1
    u2 = jnp.where(sel1, -jnp.inf, u)
    m2 = jnp.max(u2, axis=-1, keepdims=True)
    idx2 = jnp.min(jnp.where(u2 == m2, eidx, E), axis=-1, keepdims=True)
    sel2 = eidx == idx2
    denom = m1 + m2
    w = (jnp.where(sel1, m1, 0.0) + jnp.where(sel2, m2, 0.0)) / denom

    row = jax.lax.broadcasted_iota(jnp.int32, (TBLK, 1), 0) + i * TBLK
    is_moe = (row % N) >= 5
    w = jnp.where(is_moe, w, 0.0)

    acc = jax.lax.dot_general(w, be_ref[...], (((1,), (0,)), ((), ())),
                              preferred_element_type=jnp.float32)
    for e in range(E):
        eo = jax.lax.dot_general(h, we_ref[e], (((1,), (1,)), ((), ())),
                                 preferred_element_type=jnp.float32)
        acc = acc + eo * w[:, e:e + 1]

    tok = h + acc
    ob = jax.lax.dot_general(tok, wu_ref[...], (((1,), (1,)), ((), ())),
                             preferred_element_type=jnp.float32)
    out_ref[...] = ob * gamma_ref[...]


@functools.partial(jax.jit, static_argnames=())
def kernel(x, Wd, Wg, We, be, Wu, gamma):
    xf = x.reshape(B * N, IN)
    grid = (B * N // TBLK,)
    out = pl.pallas_call(
        _fused_kernel,
        grid=grid,
        in_specs=[
            pl.BlockSpec((TBLK, IN), lambda i: (i, 0)),
            pl.BlockSpec((R, IN), lambda i: (0, 0)),
            pl.BlockSpec((E, R), lambda i: (0, 0)),
            pl.BlockSpec((E, R, R), lambda i: (0, 0, 0)),
            pl.BlockSpec((E, R), lambda i: (0, 0)),
            pl.BlockSpec((OUT, R), lambda i: (0, 0)),
            pl.BlockSpec((1, OUT), lambda i: (0, 0)),
        ],
        out_specs=pl.BlockSpec((TBLK, OUT), lambda i: (i, 0)),
        out_shape=jax.ShapeDtypeStruct((B * N, OUT), jnp.float32),
    )(xf, Wd, Wg, We, be, Wu, gamma.reshape(1, OUT))
    return out.reshape(B, N, OUT)
